# baseline (device time: 703095 ns/iter reference)
import jax
import jax.numpy as jnp
from jax import lax
from jax.experimental import pallas as pl
from jax.experimental.pallas import tpu as pltpu

N_DEV = 4
HALF = 512
PIECE = 128


def kernel(x, w_mat):
    m_per, k = x.shape
    _, n_per = w_mat.shape
    m = N_DEV * m_per

    def body(x_hbm, w_hbm, out_hbm, xg_hbm,
             wbuf, xbuf, ybuf, vbuf, gbuf, amax_s,
             ag_ss, ag_rs, xl_sem, wl_sem, oc_sem, yo_sem, yp_sem,
             acp_sem, am_ss, am_rs, ep_ld, ep_st):
        my = lax.axis_index("i")
        left = lax.rem(my - 1 + N_DEV, N_DEV)
        right = lax.rem(my + 1, N_DEV)
        opp = lax.rem(my + 2, N_DEV)
        r_my = my * m_per
        r_left = left * m_per
        r_right = right * m_per
        r_opp = opp * m_per

        def rc(src, dst, s, r, dev):
            return pltpu.make_async_remote_copy(
                src_ref=src, dst_ref=dst,
                send_sem=ag_ss.at[s], recv_sem=ag_rs.at[r],
                device_id=(dev,), device_id_type=pl.DeviceIdType.MESH,
            )

        wcp = pltpu.make_async_copy(w_hbm, wbuf, wl_sem)
        wcp.start()
        ocp = pltpu.make_async_copy(x_hbm, xg_hbm.at[pl.ds(r_my, m_per)], oc_sem)
        ocp.start()

        sends = []
        for s_idx, lo, dev in [(0, 0, right), (2, HALF, right),
                               (1, HALF, left), (3, 0, left)]:
            s = rc(x_hbm.at[pl.ds(lo, HALF)], xg_hbm.at[pl.ds(r_my + lo, HALF)],
                   s_idx, s_idx, dev)
            s.start()
            sends.append(s)

        def wait_recv(row, nr, rs):
            d = rc(xg_hbm.at[pl.ds(row, nr)], xg_hbm.at[pl.ds(row, nr)],
                   0, rs, my)
            d.wait_recv()

        amax_s[0] = 0.0
        wcp.wait()
        ocp.wait()

        def half_body(i, c):
            row = jnp.where(
                i < 2,
                r_my + i * HALF,
                jnp.where(i % 2 == 0, r_left, r_right)
                + jnp.where(jnp.logical_or(i == 3, i == 4), HALF, 0),
            )

            @pl.when(i >= 2)
            def _():
                wait_recv(row, HALF, i - 2)

            @pl.when(i == 2)
            def _():
                for q in range(4):
                    rc(xg_hbm.at[pl.ds(r_left + q * PIECE, PIECE)],
                       xg_hbm.at[pl.ds(r_left + q * PIECE, PIECE)],
                       4 + q, 4 + q, right).start()

            @pl.when(i == 3)
            def _():
                for q in range(4):
                    rc(xg_hbm.at[pl.ds(r_right + HALF + q * PIECE, PIECE)],
                       xg_hbm.at[pl.ds(r_right + HALF + q * PIECE, PIECE)],
                       8 + q, 8 + q, left).start()

            xcp = pltpu.make_async_copy(
                xg_hbm.at[pl.ds(row, HALF)], xbuf.at[pl.ds(0, HALF)], xl_sem)
            xcp.start()
            xcp.wait()

            @pl.when(i >= 1)
            def _():
                pltpu.make_async_copy(
                    ybuf.at[pl.ds(0, HALF)], out_hbm.at[pl.ds(row, HALF)],
                    yo_sem).wait()

            ybuf[0:HALF, :] = jnp.dot(
                xbuf[...], wbuf[...], preferred_element_type=jnp.float32)
            amax_s[0] = jnp.maximum(
                amax_s[0], jnp.max(jnp.abs(ybuf[0:HALF, :])))
            pltpu.make_async_copy(
                ybuf.at[pl.ds(0, HALF)], out_hbm.at[pl.ds(row, HALF)],
                yo_sem).start()
            return c

        lax.fori_loop(0, 6, half_body, 0)
        pltpu.make_async_copy(
            ybuf.at[pl.ds(0, HALF)], out_hbm.at[pl.ds(0, HALF)], yo_sem).wait()

        def piece_body(q, c):
            side = lax.rem(q, 2)
            idx = q // 2
            row = r_opp + side * HALF + idx * PIECE
            wait_recv(row, PIECE, 4 + side * 4 + idx)
            xcp = pltpu.make_async_copy(
                xg_hbm.at[pl.ds(row, PIECE)], xbuf.at[pl.ds(0, PIECE)], xl_sem)
            xcp.start()
            xcp.wait()

            @pl.when(q >= 1)
            def _():
                pltpu.make_async_copy(
                    ybuf.at[pl.ds(0, PIECE)], out_hbm.at[pl.ds(row, PIECE)],
                    yp_sem).wait()

            ybuf[0:PIECE, :] = jnp.dot(
                xbuf[0:PIECE, :], wbuf[...], preferred_element_type=jnp.float32)
            amax_s[0] = jnp.maximum(
                amax_s[0], jnp.max(jnp.abs(ybuf[0:PIECE, :])))
            pltpu.make_async_copy(
                ybuf.at[pl.ds(0, PIECE)], out_hbm.at[pl.ds(row, PIECE)],
                yp_sem).start()
            return c

        lax.fori_loop(0, 8, piece_body, 0)
        pltpu.make_async_copy(
            ybuf.at[pl.ds(0, PIECE)], out_hbm.at[pl.ds(0, PIECE)], yp_sem).wait()

        for i in range(2):
            pltpu.make_async_copy(
                out_hbm.at[pl.ds(i * HALF, HALF)], ybuf.at[pl.ds(i * HALF, HALF)],
                ep_ld.at[i],
            ).start()

        vbuf[...] = jnp.full((8, 128), amax_s[0], jnp.float32)
        acp = pltpu.make_async_copy(vbuf, gbuf.at[my], acp_sem)
        acp.start()
        acp.wait()
        asends = []
        for d in range(1, N_DEV):
            tgt = lax.rem(my + d, N_DEV)
            a = pltpu.make_async_remote_copy(
                src_ref=vbuf, dst_ref=gbuf.at[my],
                send_sem=am_ss.at[d - 1], recv_sem=am_rs.at[d - 1],
                device_id=(tgt,), device_id_type=pl.DeviceIdType.MESH,
            )
            a.start()
            asends.append(a)
        for a in asends:
            a.wait_send()
        for d in range(1, N_DEV):
            src_slot = lax.rem(my - d + N_DEV, N_DEV)
            a = pltpu.make_async_remote_copy(
                src_ref=vbuf, dst_ref=gbuf.at[src_slot],
                send_sem=am_ss.at[d - 1], recv_sem=am_rs.at[d - 1],
                device_id=(my,), device_id_type=pl.DeviceIdType.MESH,
            )
            a.wait_recv()
        scale = jnp.max(gbuf[...]) / 448.0

        def quant_pair(j, c):
            for par in range(2):
                row = (2 * j + par) * HALF
                boff = par * HALF
                pltpu.make_async_copy(
                    out_hbm.at[pl.ds(row, HALF)], ybuf.at[pl.ds(boff, HALF)],
                    ep_ld.at[par],
                ).wait()
                z = jnp.clip(ybuf[boff:boff + HALF, :] / scale, -448.0, 448.0)
                ybuf[boff:boff + HALF, :] = (
                    z.astype(jnp.float8_e4m3fn).astype(jnp.float32) * scale
                )
                pltpu.make_async_copy(
                    ybuf.at[pl.ds(boff, HALF)], out_hbm.at[pl.ds(row, HALF)],
                    ep_st.at[par],
                ).start()

            @pl.when(j < (m // HALF) // 2 - 1)
            def _():
                for par in range(2):
                    row = (2 * j + par) * HALF
                    boff = par * HALF
                    pltpu.make_async_copy(
                        ybuf.at[pl.ds(boff, HALF)], out_hbm.at[pl.ds(row, HALF)],
                        ep_st.at[par],
                    ).wait()
                    pltpu.make_async_copy(
                        out_hbm.at[pl.ds(row + 2 * HALF, HALF)],
                        ybuf.at[pl.ds(boff, HALF)], ep_ld.at[par],
                    ).start()

            return c

        lax.fori_loop(0, (m // HALF) // 2, quant_pair, 0)
        for par in range(2):
            pltpu.make_async_copy(
                ybuf.at[pl.ds(par * HALF, HALF)],
                out_hbm.at[pl.ds(par * HALF, HALF)], ep_st.at[par],
            ).wait()

        for s in sends:
            s.wait_send()
        for s_idx in range(4, 12):
            rc(xg_hbm.at[pl.ds(0, PIECE)], xg_hbm.at[pl.ds(0, PIECE)],
               s_idx, 0, my).wait_send()

    out, _ = pl.pallas_call(
        body,
        out_shape=[
            jax.ShapeDtypeStruct((m, n_per), jnp.float32),
            jax.ShapeDtypeStruct((m, k), jnp.float32),
        ],
        in_specs=[
            pl.BlockSpec(memory_space=pltpu.MemorySpace.HBM),
            pl.BlockSpec(memory_space=pltpu.MemorySpace.HBM),
        ],
        out_specs=[
            pl.BlockSpec(memory_space=pltpu.MemorySpace.HBM),
            pl.BlockSpec(memory_space=pltpu.MemorySpace.HBM),
        ],
        scratch_shapes=[
            pltpu.VMEM((k, n_per), jnp.float32),
            pltpu.VMEM((HALF, k), jnp.float32),
            pltpu.VMEM((2 * HALF, n_per), jnp.float32),
            pltpu.VMEM((8, 128), jnp.float32),
            pltpu.VMEM((N_DEV, 8, 128), jnp.float32),
            pltpu.SMEM((1,), jnp.float32),
            pltpu.SemaphoreType.DMA((12,)),
            pltpu.SemaphoreType.DMA((12,)),
            pltpu.SemaphoreType.DMA,
            pltpu.SemaphoreType.DMA,
            pltpu.SemaphoreType.DMA,
            pltpu.SemaphoreType.DMA,
            pltpu.SemaphoreType.DMA,
            pltpu.SemaphoreType.DMA,
            pltpu.SemaphoreType.DMA((3,)),
            pltpu.SemaphoreType.DMA((3,)),
            pltpu.SemaphoreType.DMA((2,)),
            pltpu.SemaphoreType.DMA((2,)),
        ],
        compiler_params=pltpu.CompilerParams(
            vmem_limit_bytes=60 * 1024 * 1024,
        ),
    )(x, w_mat)
    return out


# device time: 343279 ns/iter; 2.0482x vs baseline; 2.0482x over previous
import jax

jax.config.update("jax_compilation_cache_dir", "/tmp/scband_jax_cache")
jax.config.update("jax_persistent_cache_min_compile_time_secs", 1.0)
jax.config.update("jax_persistent_cache_min_entry_size_bytes", 0)

import jax.numpy as jnp
from jax import lax
from jax.experimental import pallas as pl
from jax.experimental.pallas import tpu as pltpu

N_DEV = 4
HALF = 512
QTR = 256


def kernel(x, w_mat):
    m_per, k = x.shape
    _, n_per = w_mat.shape
    m = N_DEV * m_per

    def body(x_hbm, w_hbm, out_hbm, xg_hbm,
             wbuf, xbuf, ybuf, vbuf, gbuf, amax_s,
             ag_ss, ag_rs, xl_sem, wl_sem, yo_sems, yq_sems,
             acp_sem, am_ss, am_rs, ep_ld, ep_st):
        my = lax.axis_index("i")
        left = lax.rem(my - 1 + N_DEV, N_DEV)
        right = lax.rem(my + 1, N_DEV)
        opp = lax.rem(my + 2, N_DEV)
        r_my = my * m_per
        r_left = left * m_per
        r_right = right * m_per
        r_opp = opp * m_per

        def rc(src, dst, s, r, dev):
            return pltpu.make_async_remote_copy(
                src_ref=src, dst_ref=dst,
                send_sem=ag_ss.at[s], recv_sem=ag_rs.at[r],
                device_id=(dev,), device_id_type=pl.DeviceIdType.MESH,
            )

        wcp = pltpu.make_async_copy(w_hbm, wbuf, wl_sem)
        wcp.start()

        sends = []
        for s_idx, (lo, dev) in enumerate(
            [(0, right), (HALF, right), (HALF, left), (0, left)]
        ):
            s = rc(x_hbm.at[pl.ds(lo, HALF)], xg_hbm.at[pl.ds(r_my + lo, HALF)],
                   s_idx, s_idx, dev)
            s.start()
            sends.append(s)

        def wait_recv(row, nr, rs):
            d = rc(xg_hbm.at[pl.ds(row, nr)], xg_hbm.at[pl.ds(row, nr)],
                   0, rs, my)
            d.wait_recv()

        amax_s[0] = 0.0
        wcp.wait()

        phases = [
            (r_my, HALF, None, x_hbm.at[pl.ds(0, HALF)]),
            (r_my + HALF, HALF, None, x_hbm.at[pl.ds(HALF, HALF)]),
            (r_left, HALF, 0, None),
            (r_right + HALF, HALF, 2, None),
            (r_left + HALF, HALF, 1, None),
            (r_right, HALF, 3, None),
            (r_opp, QTR, 4, None),
            (r_opp + HALF, QTR, 6, None),
            (r_opp + QTR, QTR, 5, None),
            (r_opp + HALF + QTR, QTR, 7, None),
        ]

        youts = []
        for p, (row, nr, rs, src) in enumerate(phases):
            if rs is not None:
                wait_recv(row, nr, rs)
                src = xg_hbm.at[pl.ds(row, nr)]
            if p == 2:
                for q, s_idx in ((0, 4), (1, 5)):
                    f = rc(xg_hbm.at[pl.ds(r_left + q * QTR, QTR)],
                           xg_hbm.at[pl.ds(r_left + q * QTR, QTR)],
                           s_idx, s_idx, right)
                    f.start()
                    sends.append(f)
            if p == 3:
                for q, s_idx in ((0, 6), (1, 7)):
                    f = rc(xg_hbm.at[pl.ds(r_right + HALF + q * QTR, QTR)],
                           xg_hbm.at[pl.ds(r_right + HALF + q * QTR, QTR)],
                           s_idx, s_idx, left)
                    f.start()
                    sends.append(f)

            xcp = pltpu.make_async_copy(src, xbuf.at[pl.ds(0, nr)], xl_sem)
            xcp.start()
            xcp.wait()

            if p >= 2:
                youts[p - 2].wait()
            boff = (p % 2) * HALF
            ybuf[boff:boff + nr, :] = jnp.dot(
                xbuf[0:nr, :], wbuf[...], preferred_element_type=jnp.float32,
            )
            amax_s[0] = jnp.maximum(
                amax_s[0], jnp.max(jnp.abs(ybuf[boff:boff + nr, :]))
            )
            yo = pltpu.make_async_copy(
                ybuf.at[pl.ds(boff, nr)], out_hbm.at[pl.ds(row, nr)],
                yo_sems.at[p % 2],
            )
            yo.start()
            youts.append(yo)

        youts[-2].wait()
        youts[-1].wait()

        lds = []
        for i in range(2):
            ld = pltpu.make_async_copy(
                out_hbm.at[pl.ds(i * HALF, HALF)], ybuf.at[pl.ds(i * HALF, HALF)],
                ep_ld.at[i],
            )
            ld.start()
            lds.append(ld)

        vbuf[...] = jnp.full((8, 128), amax_s[0], jnp.float32)
        acp = pltpu.make_async_copy(vbuf, gbuf.at[my], acp_sem)
        acp.start()
        acp.wait()
        asends = []
        for d in range(1, N_DEV):
            tgt = lax.rem(my + d, N_DEV)
            a = pltpu.make_async_remote_copy(
                src_ref=vbuf, dst_ref=gbuf.at[my],
                send_sem=am_ss.at[d - 1], recv_sem=am_rs.at[d - 1],
                device_id=(tgt,), device_id_type=pl.DeviceIdType.MESH,
            )
            a.start()
            asends.append(a)
        for a in asends:
            a.wait_send()
        for d in range(1, N_DEV):
            src_slot = lax.rem(my - d + N_DEV, N_DEV)
            a = pltpu.make_async_remote_copy(
                src_ref=vbuf, dst_ref=gbuf.at[src_slot],
                send_sem=am_ss.at[d - 1], recv_sem=am_rs.at[d - 1],
                device_id=(my,), device_id_type=pl.DeviceIdType.MESH,
            )
            a.wait_recv()
        scale = jnp.max(gbuf[...]) / 448.0

        nb = m // HALF
        sts = []
        for b in range(nb):
            boff = (b % 2) * HALF
            lds[b].wait()
            z = jnp.clip(ybuf[boff:boff + HALF, :] / scale, -448.0, 448.0)
            ybuf[boff:boff + HALF, :] = (
                z.astype(jnp.float8_e4m3fn).astype(jnp.float32) * scale
            )
            st = pltpu.make_async_copy(
                ybuf.at[pl.ds(boff, HALF)], out_hbm.at[pl.ds(b * HALF, HALF)],
                ep_st.at[b % 2],
            )
            st.start()
            sts.append(st)
            if b + 2 < nb:
                st.wait()
                ld = pltpu.make_async_copy(
                    out_hbm.at[pl.ds((b + 2) * HALF, HALF)],
                    ybuf.at[pl.ds(boff, HALF)], ep_ld.at[b % 2],
                )
                ld.start()
                lds.append(ld)
        sts[-2].wait()
        sts[-1].wait()

        for s in sends:
            s.wait_send()

    out, _ = pl.pallas_call(
        body,
        out_shape=[
            jax.ShapeDtypeStruct((m, n_per), jnp.float32),
            jax.ShapeDtypeStruct((m, k), jnp.float32),
        ],
        in_specs=[
            pl.BlockSpec(memory_space=pltpu.MemorySpace.HBM),
            pl.BlockSpec(memory_space=pltpu.MemorySpace.HBM),
        ],
        out_specs=[
            pl.BlockSpec(memory_space=pltpu.MemorySpace.HBM),
            pl.BlockSpec(memory_space=pltpu.MemorySpace.HBM),
        ],
        scratch_shapes=[
            pltpu.VMEM((k, n_per), jnp.float32),
            pltpu.VMEM((HALF, k), jnp.float32),
            pltpu.VMEM((2 * HALF, n_per), jnp.float32),
            pltpu.VMEM((8, 128), jnp.float32),
            pltpu.VMEM((N_DEV, 8, 128), jnp.float32),
            pltpu.SMEM((1,), jnp.float32),
            pltpu.SemaphoreType.DMA((8,)),
            pltpu.SemaphoreType.DMA((8,)),
            pltpu.SemaphoreType.DMA,
            pltpu.SemaphoreType.DMA,
            pltpu.SemaphoreType.DMA((2,)),
            pltpu.SemaphoreType.DMA((8,)),
            pltpu.SemaphoreType.DMA,
            pltpu.SemaphoreType.DMA((3,)),
            pltpu.SemaphoreType.DMA((3,)),
            pltpu.SemaphoreType.DMA((2,)),
            pltpu.SemaphoreType.DMA((2,)),
        ],
        compiler_params=pltpu.CompilerParams(
            vmem_limit_bytes=60 * 1024 * 1024,
        ),
    )(x, w_mat)
    return out


# device time: 336848 ns/iter; 2.0873x vs baseline; 1.0191x over previous
import jax

jax.config.update("jax_compilation_cache_dir", "/tmp/scband_jax_cache")
jax.config.update("jax_persistent_cache_min_compile_time_secs", 1.0)
jax.config.update("jax_persistent_cache_min_entry_size_bytes", 0)

import jax.numpy as jnp
from jax import lax
from jax.experimental import pallas as pl
from jax.experimental.pallas import tpu as pltpu

N_DEV = 4
HALF = 512
QTR = 256
PIECE = 128


def kernel(x, w_mat):
    m_per, k = x.shape
    _, n_per = w_mat.shape
    m = N_DEV * m_per

    def body(x_hbm, w_hbm, out_hbm, xg_hbm,
             wbuf, xbuf, ybuf, vbuf, gbuf, amax_s,
             ag_ss, ag_rs, xl_sem, wl_sem, yo_sems, yq_sems,
             acp_sem, am_ss, am_rs, ep_ld, ep_st):
        my = lax.axis_index("i")
        left = lax.rem(my - 1 + N_DEV, N_DEV)
        right = lax.rem(my + 1, N_DEV)
        opp = lax.rem(my + 2, N_DEV)
        r_my = my * m_per
        r_left = left * m_per
        r_right = right * m_per
        r_opp = opp * m_per

        def rc(src, dst, s, r, dev):
            return pltpu.make_async_remote_copy(
                src_ref=src, dst_ref=dst,
                send_sem=ag_ss.at[s], recv_sem=ag_rs.at[r],
                device_id=(dev,), device_id_type=pl.DeviceIdType.MESH,
            )

        wcp = pltpu.make_async_copy(w_hbm, wbuf, wl_sem)
        wcp.start()

        sends = []
        for s_idx, (lo, dev) in enumerate(
            [(0, right), (HALF, right), (HALF, left), (0, left)]
        ):
            s = rc(x_hbm.at[pl.ds(lo, HALF)], xg_hbm.at[pl.ds(r_my + lo, HALF)],
                   s_idx, s_idx, dev)
            s.start()
            sends.append(s)

        def wait_recv(row, nr, rs):
            d = rc(xg_hbm.at[pl.ds(row, nr)], xg_hbm.at[pl.ds(row, nr)],
                   0, rs, my)
            d.wait_recv()

        amax_s[0] = 0.0
        wcp.wait()

        phases = [
            (r_my, HALF, None, x_hbm.at[pl.ds(0, HALF)]),
            (r_my + HALF, HALF, None, x_hbm.at[pl.ds(HALF, HALF)]),
            (r_left, HALF, 0, None),
            (r_right + HALF, HALF, 2, None),
            (r_left + HALF, HALF, 1, None),
            (r_right, HALF, 3, None),
        ]
        for i in range(4):
            phases.append((r_opp + i * PIECE, PIECE, 4 + i, None))
            phases.append((r_opp + HALF + i * PIECE, PIECE, 8 + i, None))

        youts = []
        for p, (row, nr, rs, src) in enumerate(phases):
            if p == 6:
                youts[4].wait()
                youts[5].wait()
            if rs is not None:
                wait_recv(row, nr, rs)
                src = xg_hbm.at[pl.ds(row, nr)]
            if p == 2:
                for q in range(4):
                    f = rc(xg_hbm.at[pl.ds(r_left + q * PIECE, PIECE)],
                           xg_hbm.at[pl.ds(r_left + q * PIECE, PIECE)],
                           4 + q, 4 + q, right)
                    f.start()
                    sends.append(f)
            if p == 3:
                for q in range(4):
                    f = rc(xg_hbm.at[pl.ds(r_right + HALF + q * PIECE, PIECE)],
                           xg_hbm.at[pl.ds(r_right + HALF + q * PIECE, PIECE)],
                           8 + q, 8 + q, left)
                    f.start()
                    sends.append(f)

            xcp = pltpu.make_async_copy(src, xbuf.at[pl.ds(0, nr)], xl_sem)
            xcp.start()
            xcp.wait()

            if p < 6:
                if p >= 2:
                    youts[p - 2].wait()
                boff = (p % 2) * HALF
                sem = yo_sems.at[p % 2]
            else:
                boff = (p - 6) * PIECE
                sem = yq_sems.at[p - 6]
            ybuf[boff:boff + nr, :] = jnp.dot(
                xbuf[0:nr, :], wbuf[...], preferred_element_type=jnp.float32,
            )
            amax_s[0] = jnp.maximum(
                amax_s[0], jnp.max(jnp.abs(ybuf[boff:boff + nr, :]))
            )
            yo = pltpu.make_async_copy(
                ybuf.at[pl.ds(boff, nr)], out_hbm.at[pl.ds(row, nr)], sem,
            )
            yo.start()
            youts.append(yo)

        for yo in youts[6:]:
            yo.wait()

        lds = []
        for i in range(2):
            ld = pltpu.make_async_copy(
                out_hbm.at[pl.ds(i * HALF, HALF)], ybuf.at[pl.ds(i * HALF, HALF)],
                ep_ld.at[i],
            )
            ld.start()
            lds.append(ld)

        vbuf[...] = jnp.full((8, 128), amax_s[0], jnp.float32)
        acp = pltpu.make_async_copy(vbuf, gbuf.at[my], acp_sem)
        acp.start()
        acp.wait()
        asends = []
        for d in range(1, N_DEV):
            tgt = lax.rem(my + d, N_DEV)
            a = pltpu.make_async_remote_copy(
                src_ref=vbuf, dst_ref=gbuf.at[my],
                send_sem=am_ss.at[d - 1], recv_sem=am_rs.at[d - 1],
                device_id=(tgt,), device_id_type=pl.DeviceIdType.MESH,
            )
            a.start()
            asends.append(a)
        for a in asends:
            a.wait_send()
        for d in range(1, N_DEV):
            src_slot = lax.rem(my - d + N_DEV, N_DEV)
            a = pltpu.make_async_remote_copy(
                src_ref=vbuf, dst_ref=gbuf.at[src_slot],
                send_sem=am_ss.at[d - 1], recv_sem=am_rs.at[d - 1],
                device_id=(my,), device_id_type=pl.DeviceIdType.MESH,
            )
            a.wait_recv()
        scale = jnp.max(gbuf[...]) / 448.0

        nb = m // HALF
        sts = []
        for b in range(nb):
            boff = (b % 2) * HALF
            lds[b].wait()
            z = jnp.clip(ybuf[boff:boff + HALF, :] / scale, -448.0, 448.0)
            ybuf[boff:boff + HALF, :] = (
                z.astype(jnp.float8_e4m3fn).astype(jnp.float32) * scale
            )
            st = pltpu.make_async_copy(
                ybuf.at[pl.ds(boff, HALF)], out_hbm.at[pl.ds(b * HALF, HALF)],
                ep_st.at[b % 2],
            )
            st.start()
            sts.append(st)
            if b + 2 < nb:
                st.wait()
                ld = pltpu.make_async_copy(
                    out_hbm.at[pl.ds((b + 2) * HALF, HALF)],
                    ybuf.at[pl.ds(boff, HALF)], ep_ld.at[b % 2],
                )
                ld.start()
                lds.append(ld)
        sts[-2].wait()
        sts[-1].wait()

        for s in sends:
            s.wait_send()

    out, _ = pl.pallas_call(
        body,
        out_shape=[
            jax.ShapeDtypeStruct((m, n_per), jnp.float32),
            jax.ShapeDtypeStruct((m, k), jnp.float32),
        ],
        in_specs=[
            pl.BlockSpec(memory_space=pltpu.MemorySpace.HBM),
            pl.BlockSpec(memory_space=pltpu.MemorySpace.HBM),
        ],
        out_specs=[
            pl.BlockSpec(memory_space=pltpu.MemorySpace.HBM),
            pl.BlockSpec(memory_space=pltpu.MemorySpace.HBM),
        ],
        scratch_shapes=[
            pltpu.VMEM((k, n_per), jnp.float32),
            pltpu.VMEM((HALF, k), jnp.float32),
            pltpu.VMEM((2 * HALF, n_per), jnp.float32),
            pltpu.VMEM((8, 128), jnp.float32),
            pltpu.VMEM((N_DEV, 8, 128), jnp.float32),
            pltpu.SMEM((1,), jnp.float32),
            pltpu.SemaphoreType.DMA((12,)),
            pltpu.SemaphoreType.DMA((12,)),
            pltpu.SemaphoreType.DMA,
            pltpu.SemaphoreType.DMA,
            pltpu.SemaphoreType.DMA((2,)),
            pltpu.SemaphoreType.DMA((8,)),
            pltpu.SemaphoreType.DMA,
            pltpu.SemaphoreType.DMA((3,)),
            pltpu.SemaphoreType.DMA((3,)),
            pltpu.SemaphoreType.DMA((2,)),
            pltpu.SemaphoreType.DMA((2,)),
        ],
        compiler_params=pltpu.CompilerParams(
            vmem_limit_bytes=60 * 1024 * 1024,
        ),
    )(x, w_mat)
    return out


# device time: 334075 ns/iter; 2.1046x vs baseline; 1.0083x over previous
import jax

jax.config.update("jax_compilation_cache_dir", "/tmp/scband_jax_cache")
jax.config.update("jax_persistent_cache_min_compile_time_secs", 1.0)
jax.config.update("jax_persistent_cache_min_entry_size_bytes", 0)

import jax.numpy as jnp
from jax import lax
from jax.experimental import pallas as pl
from jax.experimental.pallas import tpu as pltpu

N_DEV = 4
HALF = 512
QTR = 256
PIECE = 128


def kernel(x, w_mat):
    m_per, k = x.shape
    _, n_per = w_mat.shape
    m = N_DEV * m_per

    def body(x_hbm, w_hbm, out_hbm, xg_hbm,
             wbuf, xbuf, ybuf, vbuf, gbuf, amax_s,
             ag_ss, ag_rs, xl_sem, wl_sem, yo_sems, yq_sems,
             acp_sem, am_ss, am_rs, ep_ld, ep_st):
        my = lax.axis_index("i")
        left = lax.rem(my - 1 + N_DEV, N_DEV)
        right = lax.rem(my + 1, N_DEV)
        opp = lax.rem(my + 2, N_DEV)
        r_my = my * m_per
        r_left = left * m_per
        r_right = right * m_per
        r_opp = opp * m_per

        def rc(src, dst, s, r, dev):
            return pltpu.make_async_remote_copy(
                src_ref=src, dst_ref=dst,
                send_sem=ag_ss.at[s], recv_sem=ag_rs.at[r],
                device_id=(dev,), device_id_type=pl.DeviceIdType.MESH,
            )

        wcp = pltpu.make_async_copy(w_hbm, wbuf, wl_sem)
        wcp.start()

        sends = []
        for s_idx, (lo, dev) in enumerate(
            [(0, right), (HALF, right), (HALF, left), (0, left)]
        ):
            s = rc(x_hbm.at[pl.ds(lo, HALF)], xg_hbm.at[pl.ds(r_my + lo, HALF)],
                   s_idx, s_idx, dev)
            s.start()
            sends.append(s)

        def wait_recv(row, nr, rs):
            d = rc(xg_hbm.at[pl.ds(row, nr)], xg_hbm.at[pl.ds(row, nr)],
                   0, rs, my)
            d.wait_recv()

        amax_s[0] = 0.0
        wcp.wait()

        phases = [
            (r_my, HALF, None, x_hbm.at[pl.ds(0, HALF)]),
            (r_my + HALF, HALF, None, x_hbm.at[pl.ds(HALF, HALF)]),
            (r_left, HALF, 0, None),
            (r_right + HALF, HALF, 2, None),
            (r_left + HALF, HALF, 1, None),
            (r_right, HALF, 3, None),
        ]
        for i in range(4):
            phases.append((r_opp + i * PIECE, PIECE, 4 + i, None))
            phases.append((r_opp + HALF + i * PIECE, PIECE, 8 + i, None))

        youts = []
        for p, (row, nr, rs, src) in enumerate(phases):
            if p == 6:
                youts[4].wait()
                youts[5].wait()
            if rs is not None:
                wait_recv(row, nr, rs)
                src = xg_hbm.at[pl.ds(row, nr)]
            if p == 2:
                for q in range(4):
                    f = rc(xg_hbm.at[pl.ds(r_left + q * PIECE, PIECE)],
                           xg_hbm.at[pl.ds(r_left + q * PIECE, PIECE)],
                           4 + q, 4 + q, right)
                    f.start()
                    sends.append(f)
            if p == 3:
                for q in range(4):
                    f = rc(xg_hbm.at[pl.ds(r_right + HALF + q * PIECE, PIECE)],
                           xg_hbm.at[pl.ds(r_right + HALF + q * PIECE, PIECE)],
                           8 + q, 8 + q, left)
                    f.start()
                    sends.append(f)

            xcp = pltpu.make_async_copy(src, xbuf.at[pl.ds(0, nr)], xl_sem)
            xcp.start()
            xcp.wait()

            if p < 6:
                if p >= 2:
                    youts[p - 2].wait()
                boff = (p % 2) * HALF
                sem = yo_sems.at[p % 2]
            else:
                boff = (p - 6) * PIECE
                sem = yq_sems.at[p - 6]
            ybuf[boff:boff + nr, :] = jnp.dot(
                xbuf[0:nr, :], wbuf[...], preferred_element_type=jnp.float32,
            )
            amax_s[0] = jnp.maximum(
                amax_s[0], jnp.max(jnp.abs(ybuf[boff:boff + nr, :]))
            )
            yo = pltpu.make_async_copy(
                ybuf.at[pl.ds(boff, nr)], out_hbm.at[pl.ds(row, nr)], sem,
            )
            yo.start()
            youts.append(yo)

        for yo in youts[6:]:
            yo.wait()

        lds = []
        for i in range(2):
            ld = pltpu.make_async_copy(
                out_hbm.at[pl.ds(i * HALF, HALF)], ybuf.at[pl.ds(i * HALF, HALF)],
                ep_ld.at[i],
            )
            ld.start()
            lds.append(ld)

        vbuf[...] = jnp.full((8, 128), amax_s[0], jnp.float32)
        acp = pltpu.make_async_copy(vbuf, gbuf.at[my], acp_sem)
        acp.start()
        acp.wait()
        asends = []
        for d in range(1, N_DEV):
            tgt = lax.rem(my + d, N_DEV)
            a = pltpu.make_async_remote_copy(
                src_ref=vbuf, dst_ref=gbuf.at[my],
                send_sem=am_ss.at[d - 1], recv_sem=am_rs.at[d - 1],
                device_id=(tgt,), device_id_type=pl.DeviceIdType.MESH,
            )
            a.start()
            asends.append(a)
        for a in asends:
            a.wait_send()
        for d in range(1, N_DEV):
            src_slot = lax.rem(my - d + N_DEV, N_DEV)
            a = pltpu.make_async_remote_copy(
                src_ref=vbuf, dst_ref=gbuf.at[src_slot],
                send_sem=am_ss.at[d - 1], recv_sem=am_rs.at[d - 1],
                device_id=(my,), device_id_type=pl.DeviceIdType.MESH,
            )
            a.wait_recv()
        scale = jnp.max(gbuf[...]) / 448.0

        def quant_pair(j, c):
            for par in range(2):
                row = (2 * j + par) * HALF
                boff = par * HALF
                pltpu.make_async_copy(
                    out_hbm.at[pl.ds(row, HALF)], ybuf.at[pl.ds(boff, HALF)],
                    ep_ld.at[par],
                ).wait()
                z = jnp.clip(ybuf[boff:boff + HALF, :] / scale, -448.0, 448.0)
                ybuf[boff:boff + HALF, :] = (
                    z.astype(jnp.float8_e4m3fn).astype(jnp.float32) * scale
                )
                pltpu.make_async_copy(
                    ybuf.at[pl.ds(boff, HALF)], out_hbm.at[pl.ds(row, HALF)],
                    ep_st.at[par],
                ).start()

            @pl.when(j < (m // HALF) // 2 - 1)
            def _():
                for par in range(2):
                    row = (2 * j + par) * HALF
                    boff = par * HALF
                    pltpu.make_async_copy(
                        ybuf.at[pl.ds(boff, HALF)], out_hbm.at[pl.ds(row, HALF)],
                        ep_st.at[par],
                    ).wait()
                    pltpu.make_async_copy(
                        out_hbm.at[pl.ds(row + 2 * HALF, HALF)],
                        ybuf.at[pl.ds(boff, HALF)], ep_ld.at[par],
                    ).start()
            return c

        lax.fori_loop(0, (m // HALF) // 2, quant_pair, 0)
        for par in range(2):
            pltpu.make_async_copy(
                ybuf.at[pl.ds(par * HALF, HALF)],
                out_hbm.at[pl.ds(par * HALF, HALF)], ep_st.at[par],
            ).wait()

        for s in sends:
            s.wait_send()

    out, _ = pl.pallas_call(
        body,
        out_shape=[
            jax.ShapeDtypeStruct((m, n_per), jnp.float32),
            jax.ShapeDtypeStruct((m, k), jnp.float32),
        ],
        in_specs=[
            pl.BlockSpec(memory_space=pltpu.MemorySpace.HBM),
            pl.BlockSpec(memory_space=pltpu.MemorySpace.HBM),
        ],
        out_specs=[
            pl.BlockSpec(memory_space=pltpu.MemorySpace.HBM),
            pl.BlockSpec(memory_space=pltpu.MemorySpace.HBM),
        ],
        scratch_shapes=[
            pltpu.VMEM((k, n_per), jnp.float32),
            pltpu.VMEM((HALF, k), jnp.float32),
            pltpu.VMEM((2 * HALF, n_per), jnp.float32),
            pltpu.VMEM((8, 128), jnp.float32),
            pltpu.VMEM((N_DEV, 8, 128), jnp.float32),
            pltpu.SMEM((1,), jnp.float32),
            pltpu.SemaphoreType.DMA((12,)),
            pltpu.SemaphoreType.DMA((12,)),
            pltpu.SemaphoreType.DMA,
            pltpu.SemaphoreType.DMA,
            pltpu.SemaphoreType.DMA((2,)),
            pltpu.SemaphoreType.DMA((8,)),
            pltpu.SemaphoreType.DMA,
            pltpu.SemaphoreType.DMA((3,)),
            pltpu.SemaphoreType.DMA((3,)),
            pltpu.SemaphoreType.DMA((2,)),
            pltpu.SemaphoreType.DMA((2,)),
        ],
        compiler_params=pltpu.CompilerParams(
            vmem_limit_bytes=60 * 1024 * 1024,
        ),
    )(x, w_mat)
    return out


# device time: 333948 ns/iter; 2.1054x vs baseline; 1.0004x over previous
import jax

jax.config.update("jax_compilation_cache_dir", "/tmp/scband_jax_cache")
jax.config.update("jax_persistent_cache_min_compile_time_secs", 1.0)
jax.config.update("jax_persistent_cache_min_entry_size_bytes", 0)

import jax.numpy as jnp
from jax import lax
from jax.experimental import pallas as pl
from jax.experimental.pallas import tpu as pltpu

N_DEV = 4
HALF = 512
PIECE = 128


def kernel(x, w_mat):
    m_per, k = x.shape
    _, n_per = w_mat.shape
    m = N_DEV * m_per

    def body(x_hbm, w_hbm, out_hbm, xg_hbm,
             wbuf, xbuf, ybuf, vbuf, gbuf, amax_s,
             ag_ss, ag_rs, xl_sem, wl_sem, yo_sems, yq_sems,
             acp_sem, am_ss, am_rs, ep_ld, ep_st):
        my = lax.axis_index("i")
        left = lax.rem(my - 1 + N_DEV, N_DEV)
        right = lax.rem(my + 1, N_DEV)
        opp = lax.rem(my + 2, N_DEV)
        r_my = my * m_per
        r_left = left * m_per
        r_right = right * m_per
        r_opp = opp * m_per

        def rc(src, dst, s, r, dev):
            return pltpu.make_async_remote_copy(
                src_ref=src, dst_ref=dst,
                send_sem=ag_ss.at[s], recv_sem=ag_rs.at[r],
                device_id=(dev,), device_id_type=pl.DeviceIdType.MESH,
            )

        wcp = pltpu.make_async_copy(w_hbm, wbuf, wl_sem)
        wcp.start()

        sends = []
        for s_idx, (lo, dev) in enumerate(
            [(0, right), (HALF, right), (HALF, left), (0, left)]
        ):
            s = rc(x_hbm.at[pl.ds(lo, HALF)], xg_hbm.at[pl.ds(r_my + lo, HALF)],
                   s_idx, s_idx, dev)
            s.start()
            sends.append(s)

        def wait_recv(row, nr, rs):
            d = rc(xg_hbm.at[pl.ds(row, nr)], xg_hbm.at[pl.ds(row, nr)],
                   0, rs, my)
            d.wait_recv()

        amax_s[0] = 0.0
        wcp.wait()

        phases = [
            (r_my, HALF, None, x_hbm.at[pl.ds(0, HALF)]),
            (r_my + HALF, HALF, None, x_hbm.at[pl.ds(HALF, HALF)]),
            (r_left, HALF, 0, None),
            (r_right + HALF, HALF, 2, None),
            (r_left + HALF, HALF, 1, None),
            (r_right, HALF, 3, None),
        ]
        for i in range(4):
            phases.append((r_opp + i * PIECE, PIECE, 4 + i, None))
            phases.append((r_opp + HALF + i * PIECE, PIECE, 8 + i, None))

        youts = []
        for p, (row, nr, rs, src) in enumerate(phases):
            if p == 6:
                youts[4].wait()
                youts[5].wait()
            if rs is not None:
                wait_recv(row, nr, rs)
                src = xg_hbm.at[pl.ds(row, nr)]
            if p == 2:
                for q in range(4):
                    f = rc(xg_hbm.at[pl.ds(r_left + q * PIECE, PIECE)],
                           xg_hbm.at[pl.ds(r_left + q * PIECE, PIECE)],
                           4 + q, 4 + q, right)
                    f.start()
                    sends.append(f)
            if p == 3:
                for q in range(4):
                    f = rc(xg_hbm.at[pl.ds(r_right + HALF + q * PIECE, PIECE)],
                           xg_hbm.at[pl.ds(r_right + HALF + q * PIECE, PIECE)],
                           8 + q, 8 + q, left)
                    f.start()
                    sends.append(f)

            xcp = pltpu.make_async_copy(src, xbuf.at[pl.ds(0, nr)], xl_sem)
            xcp.start()
            xcp.wait()

            if p < 6:
                if p >= 2:
                    youts[p - 2].wait()
                boff = (p % 2) * HALF
                sem = yo_sems.at[p % 2]
            else:
                boff = (p - 6) * PIECE
                sem = yq_sems.at[p - 6]
            ybuf[boff:boff + nr, :] = jnp.dot(
                xbuf[0:nr, :], wbuf[...], preferred_element_type=jnp.float32,
            )
            amax_s[0] = jnp.maximum(
                amax_s[0], jnp.max(jnp.abs(ybuf[boff:boff + nr, :]))
            )
            yo = pltpu.make_async_copy(
                ybuf.at[pl.ds(boff, nr)], out_hbm.at[pl.ds(row, nr)], sem,
            )
            yo.start()
            youts.append(yo)

        for yo in youts[6:]:
            yo.wait()

        lds = []
        for i in range(2):
            ld = pltpu.make_async_copy(
                out_hbm.at[pl.ds(i * HALF, HALF)], ybuf.at[pl.ds(i * HALF, HALF)],
                ep_ld.at[i],
            )
            ld.start()
            lds.append(ld)

        vbuf[...] = jnp.full((8, 128), amax_s[0], jnp.float32)
        acp = pltpu.make_async_copy(vbuf, gbuf.at[my], acp_sem)
        acp.start()
        acp.wait()
        asends = []
        for d in range(1, N_DEV):
            tgt = lax.rem(my + d, N_DEV)
            a = pltpu.make_async_remote_copy(
                src_ref=vbuf, dst_ref=gbuf.at[my],
                send_sem=am_ss.at[d - 1], recv_sem=am_rs.at[d - 1],
                device_id=(tgt,), device_id_type=pl.DeviceIdType.MESH,
            )
            a.start()
            asends.append(a)
        for a in asends:
            a.wait_send()
        for d in range(1, N_DEV):
            src_slot = lax.rem(my - d + N_DEV, N_DEV)
            a = pltpu.make_async_remote_copy(
                src_ref=vbuf, dst_ref=gbuf.at[src_slot],
                send_sem=am_ss.at[d - 1], recv_sem=am_rs.at[d - 1],
                device_id=(my,), device_id_type=pl.DeviceIdType.MESH,
            )
            a.wait_recv()
        scale = jnp.max(gbuf[...]) / 448.0

        def quant_pair(j, c):
            for par in range(2):
                row = (2 * j + par) * HALF
                boff = par * HALF
                pltpu.make_async_copy(
                    out_hbm.at[pl.ds(row, HALF)], ybuf.at[pl.ds(boff, HALF)],
                    ep_ld.at[par],
                ).wait()
                z = jnp.clip(ybuf[boff:boff + HALF, :] / scale, -448.0, 448.0)
                ybuf[boff:boff + HALF, :] = (
                    z.astype(jnp.float8_e4m3fn).astype(jnp.float32) * scale
                )
                pltpu.make_async_copy(
                    ybuf.at[pl.ds(boff, HALF)], out_hbm.at[pl.ds(row, HALF)],
                    ep_st.at[par],
                ).start()

            @pl.when(j < (m // HALF) // 2 - 1)
            def _():
                for par in range(2):
                    row = (2 * j + par) * HALF
                    boff = par * HALF
                    pltpu.make_async_copy(
                        ybuf.at[pl.ds(boff, HALF)], out_hbm.at[pl.ds(row, HALF)],
                        ep_st.at[par],
                    ).wait()
                    pltpu.make_async_copy(
                        out_hbm.at[pl.ds(row + 2 * HALF, HALF)],
                        ybuf.at[pl.ds(boff, HALF)], ep_ld.at[par],
                    ).start()
            return c

        lax.fori_loop(0, (m // HALF) // 2, quant_pair, 0)
        for par in range(2):
            pltpu.make_async_copy(
                ybuf.at[pl.ds(par * HALF, HALF)],
                out_hbm.at[pl.ds(par * HALF, HALF)], ep_st.at[par],
            ).wait()

        for s in sends:
            s.wait_send()

    out, _ = pl.pallas_call(
        body,
        out_shape=[
            jax.ShapeDtypeStruct((m, n_per), jnp.float32),
            jax.ShapeDtypeStruct((m, k), jnp.float32),
        ],
        in_specs=[
            pl.BlockSpec(memory_space=pltpu.MemorySpace.HBM),
            pl.BlockSpec(memory_space=pltpu.MemorySpace.HBM),
        ],
        out_specs=[
            pl.BlockSpec(memory_space=pltpu.MemorySpace.HBM),
            pl.BlockSpec(memory_space=pltpu.MemorySpace.HBM),
        ],
        scratch_shapes=[
            pltpu.VMEM((k, n_per), jnp.float32),
            pltpu.VMEM((HALF, k), jnp.float32),
            pltpu.VMEM((2 * HALF, n_per), jnp.float32),
            pltpu.VMEM((8, 128), jnp.float32),
            pltpu.VMEM((N_DEV, 8, 128), jnp.float32),
            pltpu.SMEM((1,), jnp.float32),
            pltpu.SemaphoreType.DMA((12,)),
            pltpu.SemaphoreType.DMA((12,)),
            pltpu.SemaphoreType.DMA,
            pltpu.SemaphoreType.DMA,
            pltpu.SemaphoreType.DMA((2,)),
            pltpu.SemaphoreType.DMA((8,)),
            pltpu.SemaphoreType.DMA,
            pltpu.SemaphoreType.DMA((3,)),
            pltpu.SemaphoreType.DMA((3,)),
            pltpu.SemaphoreType.DMA((2,)),
            pltpu.SemaphoreType.DMA((2,)),
        ],
        compiler_params=pltpu.CompilerParams(
            vmem_limit_bytes=60 * 1024 * 1024,
        ),
    )(x, w_mat)
    return out
